# baseline (speedup 1.0000x reference)
import jax
import jax.numpy as jnp
from jax import lax
from jax.experimental import pallas as pl
from jax.experimental.pallas import tpu as pltpu
from jax.experimental.pallas import tpu_sc as plsc

B, L, D = 16384, 200, 32
NC, NU = 100000, 100000

NCORES, NSUB, LANES = 2, 16, 16
NW = NCORES * NSUB
RPW = B // NW
CB = 8
NCHUNK = RPW // CB
IDX_N = CB * L
WIN = 128
NWIN_MAX = -(-IDX_N // WIN)
CAP = NWIN_MAX * WIN
BL = B * L


def _splat(v):
    return jnp.full((LANES,), 0, jnp.int32) + v


def _to_scalar(vec):
    return lax.reduce_max(vec, axes=(0,))


def _sc_body(idx_hbm, nu_hbm, tab_hbm, out_hbm,
             ctraw0, utraw0, ccv0, ctraw1, utraw1, ccv1,
             nuv,
             ctc0, utc0, starts0, ctrows0, utrows0,
             ctc1, utc1, starts1, ctrows1, utrows1,
             outv0, outv1,
             semi0, semi1, semg0, semg1, semo0, semo1):
    wid = lax.axis_index("c") * NSUB + lax.axis_index("s")
    row0 = wid * RPW
    lane = lax.iota(jnp.int32, LANES)
    lane0 = lane == 0

    pltpu.sync_copy(nu_hbm.at[pl.ds(row0, RPW)], nuv)

    @pl.loop(0, CAP // LANES)
    def _fill(i):
        z = jnp.zeros((LANES,), jnp.int32)
        ctc0[pl.ds(i * LANES, LANES)] = z
        utc0[pl.ds(i * LANES, LANES)] = z
        ctc1[pl.ds(i * LANES, LANES)] = z
        utc1[pl.ds(i * LANES, LANES)] = z

    def fire_idx(ci, bufs, sem):
        ctraw, utraw, ccv = bufs
        base = row0 * L + ci * IDX_N
        pltpu.async_copy(idx_hbm.at[pl.ds(base, IDX_N)], ctraw, sem)
        pltpu.async_copy(idx_hbm.at[pl.ds(BL + base, IDX_N)], utraw, sem)
        pltpu.async_copy(idx_hbm.at[pl.ds(2 * BL + base, IDX_N)], ccv, sem)

    def wait_idx(ci, bufs, sem):
        ctraw, utraw, ccv = bufs
        base = row0 * L + ci * IDX_N
        pltpu.make_async_copy(idx_hbm.at[pl.ds(base, IDX_N)], ctraw,
                              sem).wait()
        pltpu.make_async_copy(idx_hbm.at[pl.ds(BL + base, IDX_N)], utraw,
                              sem).wait()
        pltpu.make_async_copy(idx_hbm.at[pl.ds(2 * BL + base, IDX_N)], ccv,
                              sem).wait()

    def compact(ci, bufs, g):
        ctraw, utraw, ccv = bufs
        ctc, utc, starts = g[0], g[1], g[2]

        def row_compact(b, off):
            plsc.store_scatter(starts, [_splat(b)], off, mask=lane0)
            nusplat = plsc.load_gather(nuv, [_splat(ci * CB + b)])
            for gi in range(13):
                l0 = gi * 16
                base = b * L + l0
                lvec = lane + l0
                cc16 = ccv[pl.ds(base, LANES)]
                valid = (lvec < nusplat) & (cc16 == 0)
                if gi == 12:
                    valid = valid & (lane < 8)
                cs = plsc.cumsum(valid.astype(jnp.int32))
                dst = off + cs - 1
                plsc.store_scatter(ctc, [dst], ctraw[pl.ds(base, LANES)],
                                   mask=valid)
                plsc.store_scatter(utc, [dst],
                                   utraw[pl.ds(base, LANES)] + NC,
                                   mask=valid)
                off = off + plsc.all_reduce_population_count(valid)
            return off

        off = lax.fori_loop(0, CB, row_compact,
                            jnp.zeros((LANES,), jnp.int32))
        plsc.store_scatter(starts, [_splat(CB)], off, mask=lane0)

    def counts(g):
        total = _to_scalar(plsc.load_gather(g[2], [_splat(CB)]))
        nfull = total // WIN
        ntail = (total - nfull * WIN + 31) // 32
        return nfull, ntail

    def fire_g(g, semg):
        ctc, utc, ctrows, utrows = g[0], g[1], g[3], g[4]
        nfull, ntail = counts(g)

        @pl.loop(0, nfull)
        def _fire(j):
            sl = pl.ds(j * WIN, WIN)
            pltpu.async_copy(tab_hbm.at[ctc.at[sl]], ctrows.at[sl], semg)
            pltpu.async_copy(tab_hbm.at[utc.at[sl]], utrows.at[sl], semg)

        @pl.loop(0, ntail)
        def _firet(j):
            sl = pl.ds(nfull * WIN + j * 32, 32)
            pltpu.async_copy(tab_hbm.at[ctc.at[sl]], ctrows.at[sl], semg)
            pltpu.async_copy(tab_hbm.at[utc.at[sl]], utrows.at[sl], semg)

    def drain_g(g, semg):
        ctc, utc, ctrows, utrows = g[0], g[1], g[3], g[4]
        nfull, ntail = counts(g)

        @pl.loop(0, nfull)
        def _drain(j):
            sl = pl.ds(j * WIN, WIN)
            pltpu.make_async_copy(tab_hbm.at[ctc.at[sl]],
                                  ctrows.at[sl], semg).wait()
            pltpu.make_async_copy(tab_hbm.at[utc.at[sl]],
                                  utrows.at[sl], semg).wait()

        @pl.loop(0, ntail)
        def _draint(j):
            sl = pl.ds(nfull * WIN + j * 32, 32)
            pltpu.make_async_copy(tab_hbm.at[ctc.at[sl]],
                                  ctrows.at[sl], semg).wait()
            pltpu.make_async_copy(tab_hbm.at[utc.at[sl]],
                                  utrows.at[sl], semg).wait()

    def accum_out(ci, g, outv, semo):
        starts, ctrows, utrows = g[2], g[3], g[4]

        @pl.when(ci >= 2)
        def _drain_out():
            back = pl.ds(row0 + (ci - 2) * CB, CB)
            pltpu.make_async_copy(outv, out_hbm.at[back], semo).wait()

        @pl.loop(0, CB)
        def _accum(b):
            s = _to_scalar(plsc.load_gather(starts, [_splat(b)]))
            e = _to_scalar(plsc.load_gather(starts, [_splat(b + 1)]))

            def lbody(l, accs):
                a0, a1 = accs
                c2 = ctrows[l, pl.ds(0, 2 * LANES)]
                u2 = utrows[l, pl.ds(0, 2 * LANES)]
                pa, pb = plsc.unpack(c2 * u2,
                                     format=plsc.PackFormat.INTERLEAVED)
                return (a0 + pa, a1 + pb)

            z = jnp.zeros((LANES,), jnp.float32)
            a0, a1 = lax.fori_loop(s, e, lbody, (z, z))
            outv[b, pl.ds(0, LANES)] = a0
            outv[b, pl.ds(LANES, LANES)] = a1

        pltpu.async_copy(outv, out_hbm.at[pl.ds(row0 + ci * CB, CB)], semo)

    i0 = (ctraw0, utraw0, ccv0)
    i1 = (ctraw1, utraw1, ccv1)
    g0 = (ctc0, utc0, starts0, ctrows0, utrows0)
    g1 = (ctc1, utc1, starts1, ctrows1, utrows1)

    fire_idx(0, i0, semi0)
    fire_idx(1, i1, semi1)
    wait_idx(0, i0, semi0)
    compact(0, i0, g0)
    fire_g(g0, semg0)

    @pl.loop(0, NCHUNK, step=2)
    def _chunks(ci):
        wait_idx(ci + 1, i1, semi1)
        compact(ci + 1, i1, g1)
        fire_g(g1, semg1)
        drain_g(g0, semg0)

        @pl.when(ci + 2 < NCHUNK)
        def _pf0():
            fire_idx(ci + 2, i0, semi0)

        accum_out(ci, g0, outv0, semo0)

        @pl.when(ci + 2 < NCHUNK)
        def _next0():
            wait_idx(ci + 2, i0, semi0)
            compact(ci + 2, i0, g0)
            fire_g(g0, semg0)

        drain_g(g1, semg1)

        @pl.when(ci + 3 < NCHUNK)
        def _pf1():
            fire_idx(ci + 3, i1, semi1)

        accum_out(ci + 1, g1, outv1, semo1)

    lastrow0 = pl.ds(row0 + (NCHUNK - 2) * CB, CB)
    lastrow1 = pl.ds(row0 + (NCHUNK - 1) * CB, CB)
    pltpu.make_async_copy(outv0, out_hbm.at[lastrow0], semo0).wait()
    pltpu.make_async_copy(outv1, out_hbm.at[lastrow1], semo1).wait()


def kernel(ctype, ccont, utype, num_unit, ctype_w, utype_w, ccont_w):
    del ccont_w
    idxcat = jnp.concatenate([ctype.reshape(BL), utype.reshape(BL),
                              ccont.reshape(BL)])
    perm = jnp.arange(D).reshape(2, D // 2).T.reshape(D)
    tabcat = jnp.concatenate([ctype_w[:, perm],
                              utype_w[:, perm]]).astype(jnp.bfloat16)

    mesh = plsc.VectorSubcoreMesh(
        core_axis_name="c", subcore_axis_name="s",
        num_cores=NCORES, num_subcores=NSUB)
    cp = pltpu.CompilerParams(
        needs_layout_passes=False, use_tc_tiling_on_sc=False)
    run = pl.kernel(
        _sc_body,
        out_type=jax.ShapeDtypeStruct((B, D), jnp.float32),
        mesh=mesh,
        compiler_params=cp,
        scratch_types=[
            pltpu.VMEM((IDX_N,), jnp.int32),
            pltpu.VMEM((IDX_N,), jnp.int32),
            pltpu.VMEM((IDX_N,), jnp.int32),
            pltpu.VMEM((IDX_N,), jnp.int32),
            pltpu.VMEM((IDX_N,), jnp.int32),
            pltpu.VMEM((IDX_N,), jnp.int32),
            pltpu.VMEM((RPW,), jnp.int32),
            pltpu.VMEM((CAP,), jnp.int32),
            pltpu.VMEM((CAP,), jnp.int32),
            pltpu.VMEM((16,), jnp.int32),
            pltpu.VMEM((CAP, D), jnp.bfloat16),
            pltpu.VMEM((CAP, D), jnp.bfloat16),
            pltpu.VMEM((CAP,), jnp.int32),
            pltpu.VMEM((CAP,), jnp.int32),
            pltpu.VMEM((16,), jnp.int32),
            pltpu.VMEM((CAP, D), jnp.bfloat16),
            pltpu.VMEM((CAP, D), jnp.bfloat16),
            pltpu.VMEM((CB, D), jnp.float32),
            pltpu.VMEM((CB, D), jnp.float32),
            pltpu.SemaphoreType.DMA,
            pltpu.SemaphoreType.DMA,
            pltpu.SemaphoreType.DMA,
            pltpu.SemaphoreType.DMA,
            pltpu.SemaphoreType.DMA,
            pltpu.SemaphoreType.DMA,
        ],
    )
    return run(idxcat, num_unit, tabcat)


# speedup vs baseline: 1.0792x; 1.0792x over previous
import jax
import jax.numpy as jnp
from jax import lax
from jax.experimental import pallas as pl
from jax.experimental.pallas import tpu as pltpu
from jax.experimental.pallas import tpu_sc as plsc

B, L, D = 16384, 200, 32
NC, NU = 100000, 100000

NCORES, NSUB, LANES = 2, 16, 16
NW = NCORES * NSUB
RPW = B // NW
CB = 8
NCHUNK = RPW // CB
IDX_N = CB * L
WIN = 128
NWIN_MAX = -(-IDX_N // WIN)
CAP = NWIN_MAX * WIN
BL = B * L


def _splat(v):
    return jnp.full((LANES,), 0, jnp.int32) + v


def _to_scalar(vec):
    return lax.reduce_max(vec, axes=(0,))


def _sc_body(ct_hbm, ut_hbm, cc_hbm, nu_hbm, tab_hbm, out_hbm,
             ctraw0, utraw0, ccv0, ctraw1, utraw1, ccv1,
             nuv,
             ctc0, utc0, starts0, ctrows0, utrows0,
             ctc1, utc1, starts1, ctrows1, utrows1,
             outv0, outv1,
             semi0, semi1, semg0, semg1, semo0, semo1):
    wid = lax.axis_index("c") * NSUB + lax.axis_index("s")
    row0 = wid * RPW
    lane = lax.iota(jnp.int32, LANES)
    lane0 = lane == 0

    pltpu.sync_copy(nu_hbm.at[pl.ds(row0, RPW)], nuv)

    @pl.loop(0, CAP // LANES)
    def _fill(i):
        z = jnp.zeros((LANES,), jnp.int32)
        ctc0[pl.ds(i * LANES, LANES)] = z
        utc0[pl.ds(i * LANES, LANES)] = z
        ctc1[pl.ds(i * LANES, LANES)] = z
        utc1[pl.ds(i * LANES, LANES)] = z

    def fire_idx(ci, bufs, sem):
        ctraw, utraw, ccv = bufs
        sl = pl.ds(row0 * L + ci * IDX_N, IDX_N)
        pltpu.async_copy(ct_hbm.at[sl], ctraw, sem)
        pltpu.async_copy(ut_hbm.at[sl], utraw, sem)
        pltpu.async_copy(cc_hbm.at[sl], ccv, sem)

    def wait_idx(ci, bufs, sem):
        ctraw, utraw, ccv = bufs
        sl = pl.ds(row0 * L + ci * IDX_N, IDX_N)
        pltpu.make_async_copy(ct_hbm.at[sl], ctraw, sem).wait()
        pltpu.make_async_copy(ut_hbm.at[sl], utraw, sem).wait()
        pltpu.make_async_copy(cc_hbm.at[sl], ccv, sem).wait()

    def compact(ci, bufs, g):
        ctraw, utraw, ccv = bufs
        ctc, utc, starts = g[0], g[1], g[2]

        def row_compact(b, off):
            plsc.store_scatter(starts, [_splat(b)], off, mask=lane0)
            nusplat = plsc.load_gather(nuv, [_splat(ci * CB + b)])
            for gi in range(13):
                l0 = gi * 16
                base = b * L + l0
                lvec = lane + l0
                cc16 = ccv[pl.ds(base, LANES)]
                valid = (lvec < nusplat) & (cc16 == 0)
                if gi == 12:
                    valid = valid & (lane < 8)
                cs = plsc.cumsum(valid.astype(jnp.int32))
                dst = off + cs - 1
                plsc.store_scatter(ctc, [dst], ctraw[pl.ds(base, LANES)],
                                   mask=valid)
                plsc.store_scatter(utc, [dst],
                                   utraw[pl.ds(base, LANES)] + NC,
                                   mask=valid)
                off = off + plsc.all_reduce_population_count(valid)
            return off

        off = lax.fori_loop(0, CB, row_compact,
                            jnp.zeros((LANES,), jnp.int32))
        plsc.store_scatter(starts, [_splat(CB)], off, mask=lane0)

    def counts(g):
        total = _to_scalar(plsc.load_gather(g[2], [_splat(CB)]))
        nfull = total // WIN
        ntail = (total - nfull * WIN + 31) // 32
        return nfull, ntail

    def fire_g(g, semg):
        ctc, utc, ctrows, utrows = g[0], g[1], g[3], g[4]
        nfull, ntail = counts(g)

        @pl.loop(0, nfull)
        def _fire(j):
            sl = pl.ds(j * WIN, WIN)
            pltpu.async_copy(tab_hbm.at[ctc.at[sl]], ctrows.at[sl], semg)
            pltpu.async_copy(tab_hbm.at[utc.at[sl]], utrows.at[sl], semg)

        @pl.loop(0, ntail)
        def _firet(j):
            sl = pl.ds(nfull * WIN + j * 32, 32)
            pltpu.async_copy(tab_hbm.at[ctc.at[sl]], ctrows.at[sl], semg)
            pltpu.async_copy(tab_hbm.at[utc.at[sl]], utrows.at[sl], semg)

    def drain_g(g, semg):
        ctc, utc, ctrows, utrows = g[0], g[1], g[3], g[4]
        nfull, ntail = counts(g)

        @pl.loop(0, nfull)
        def _drain(j):
            sl = pl.ds(j * WIN, WIN)
            pltpu.make_async_copy(tab_hbm.at[ctc.at[sl]],
                                  ctrows.at[sl], semg).wait()
            pltpu.make_async_copy(tab_hbm.at[utc.at[sl]],
                                  utrows.at[sl], semg).wait()

        @pl.loop(0, ntail)
        def _draint(j):
            sl = pl.ds(nfull * WIN + j * 32, 32)
            pltpu.make_async_copy(tab_hbm.at[ctc.at[sl]],
                                  ctrows.at[sl], semg).wait()
            pltpu.make_async_copy(tab_hbm.at[utc.at[sl]],
                                  utrows.at[sl], semg).wait()

    def accum_out(ci, g, outv, semo):
        starts, ctrows, utrows = g[2], g[3], g[4]

        @pl.when(ci >= 2)
        def _drain_out():
            back = pl.ds(row0 + (ci - 2) * CB, CB)
            pltpu.make_async_copy(outv, out_hbm.at[back], semo).wait()

        @pl.loop(0, CB)
        def _accum(b):
            s = _to_scalar(plsc.load_gather(starts, [_splat(b)]))
            e = _to_scalar(plsc.load_gather(starts, [_splat(b + 1)]))

            def lbody(l, accs):
                a0, a1 = accs
                c2 = ctrows[l, pl.ds(0, 2 * LANES)]
                u2 = utrows[l, pl.ds(0, 2 * LANES)]
                pa, pb = plsc.unpack(c2 * u2,
                                     format=plsc.PackFormat.INTERLEAVED)
                return (a0 + pa, a1 + pb)

            z = jnp.zeros((LANES,), jnp.float32)
            a0, a1 = lax.fori_loop(s, e, lbody, (z, z))
            outv[b, pl.ds(0, LANES)] = a0
            outv[b, pl.ds(LANES, LANES)] = a1

        pltpu.async_copy(outv, out_hbm.at[pl.ds(row0 + ci * CB, CB)], semo)

    i0 = (ctraw0, utraw0, ccv0)
    i1 = (ctraw1, utraw1, ccv1)
    g0 = (ctc0, utc0, starts0, ctrows0, utrows0)
    g1 = (ctc1, utc1, starts1, ctrows1, utrows1)

    fire_idx(0, i0, semi0)
    fire_idx(1, i1, semi1)
    wait_idx(0, i0, semi0)
    compact(0, i0, g0)
    fire_g(g0, semg0)

    @pl.loop(0, NCHUNK, step=2)
    def _chunks(ci):
        wait_idx(ci + 1, i1, semi1)
        compact(ci + 1, i1, g1)
        fire_g(g1, semg1)
        drain_g(g0, semg0)

        @pl.when(ci + 2 < NCHUNK)
        def _pf0():
            fire_idx(ci + 2, i0, semi0)

        accum_out(ci, g0, outv0, semo0)

        @pl.when(ci + 2 < NCHUNK)
        def _next0():
            wait_idx(ci + 2, i0, semi0)
            compact(ci + 2, i0, g0)
            fire_g(g0, semg0)

        drain_g(g1, semg1)

        @pl.when(ci + 3 < NCHUNK)
        def _pf1():
            fire_idx(ci + 3, i1, semi1)

        accum_out(ci + 1, g1, outv1, semo1)

    lastrow0 = pl.ds(row0 + (NCHUNK - 2) * CB, CB)
    lastrow1 = pl.ds(row0 + (NCHUNK - 1) * CB, CB)
    pltpu.make_async_copy(outv0, out_hbm.at[lastrow0], semo0).wait()
    pltpu.make_async_copy(outv1, out_hbm.at[lastrow1], semo1).wait()


def kernel(ctype, ccont, utype, num_unit, ctype_w, utype_w, ccont_w):
    del ccont_w
    ct1 = ctype.reshape(BL)
    ut1 = utype.reshape(BL)
    cc1 = ccont.reshape(BL)
    perm = jnp.arange(D).reshape(2, D // 2).T.reshape(D)
    tabcat = jnp.concatenate([ctype_w[:, perm],
                              utype_w[:, perm]]).astype(jnp.bfloat16)

    mesh = plsc.VectorSubcoreMesh(
        core_axis_name="c", subcore_axis_name="s",
        num_cores=NCORES, num_subcores=NSUB)
    cp = pltpu.CompilerParams(
        needs_layout_passes=False, use_tc_tiling_on_sc=False)
    run = pl.kernel(
        _sc_body,
        out_type=jax.ShapeDtypeStruct((B, D), jnp.float32),
        mesh=mesh,
        compiler_params=cp,
        scratch_types=[
            pltpu.VMEM((IDX_N,), jnp.int32),
            pltpu.VMEM((IDX_N,), jnp.int32),
            pltpu.VMEM((IDX_N,), jnp.int32),
            pltpu.VMEM((IDX_N,), jnp.int32),
            pltpu.VMEM((IDX_N,), jnp.int32),
            pltpu.VMEM((IDX_N,), jnp.int32),
            pltpu.VMEM((RPW,), jnp.int32),
            pltpu.VMEM((CAP,), jnp.int32),
            pltpu.VMEM((CAP,), jnp.int32),
            pltpu.VMEM((16,), jnp.int32),
            pltpu.VMEM((CAP, D), jnp.bfloat16),
            pltpu.VMEM((CAP, D), jnp.bfloat16),
            pltpu.VMEM((CAP,), jnp.int32),
            pltpu.VMEM((CAP,), jnp.int32),
            pltpu.VMEM((16,), jnp.int32),
            pltpu.VMEM((CAP, D), jnp.bfloat16),
            pltpu.VMEM((CAP, D), jnp.bfloat16),
            pltpu.VMEM((CB, D), jnp.float32),
            pltpu.VMEM((CB, D), jnp.float32),
            pltpu.SemaphoreType.DMA,
            pltpu.SemaphoreType.DMA,
            pltpu.SemaphoreType.DMA,
            pltpu.SemaphoreType.DMA,
            pltpu.SemaphoreType.DMA,
            pltpu.SemaphoreType.DMA,
        ],
    )
    return run(ct1, ut1, cc1, num_unit, tabcat)


# speedup vs baseline: 1.1795x; 1.0930x over previous
import jax
import jax.numpy as jnp
from jax import lax
from jax.experimental import pallas as pl
from jax.experimental.pallas import tpu as pltpu
from jax.experimental.pallas import tpu_sc as plsc

B, L, D = 16384, 200, 32
NC, NU = 100000, 100000

NCORES, NSUB, LANES = 2, 16, 16
NW = NCORES * NSUB
RPW = B // NW
CB = 8
NCHUNK = RPW // CB
IDX_N = CB * L
WIN = 128
NWIN_MAX = -(-IDX_N // WIN)
CAP = NWIN_MAX * WIN
BL = B * L
ZERO_IDX = NC


def _splat(v):
    return jnp.full((LANES,), 0, jnp.int32) + v


def _to_scalar(vec):
    return lax.reduce_max(vec, axes=(0,))


def _sc_body(ct_hbm, ut_hbm, cc_hbm, nu_hbm, ctw_hbm, utw_hbm, out_hbm,
             ctraw0, utraw0, ccv0, ctraw1, utraw1, ccv1,
             nuv,
             ctc0, utc0, starts0, ctrows0, utrows0,
             ctc1, utc1, starts1, ctrows1, utrows1,
             outv0, outv1,
             semi0, semi1, semg0, semg1, semo0, semo1):
    wid = lax.axis_index("c") * NSUB + lax.axis_index("s")
    row0 = wid * RPW
    lane = lax.iota(jnp.int32, LANES)
    lane0 = lane == 0

    pltpu.sync_copy(nu_hbm.at[pl.ds(row0, RPW)], nuv)

    @pl.loop(0, CAP // LANES)
    def _fill(i):
        z = jnp.zeros((LANES,), jnp.int32)
        ctc0[pl.ds(i * LANES, LANES)] = z
        utc0[pl.ds(i * LANES, LANES)] = z
        ctc1[pl.ds(i * LANES, LANES)] = z
        utc1[pl.ds(i * LANES, LANES)] = z

    def fire_idx(ci, bufs, sem):
        ctraw, utraw, ccv = bufs
        sl = pl.ds(row0 * L + ci * IDX_N, IDX_N)
        pltpu.async_copy(ct_hbm.at[sl], ctraw, sem)
        pltpu.async_copy(ut_hbm.at[sl], utraw, sem)
        pltpu.async_copy(cc_hbm.at[sl], ccv, sem)

    def wait_idx(ci, bufs, sem):
        ctraw, utraw, ccv = bufs
        sl = pl.ds(row0 * L + ci * IDX_N, IDX_N)
        pltpu.make_async_copy(ct_hbm.at[sl], ctraw, sem).wait()
        pltpu.make_async_copy(ut_hbm.at[sl], utraw, sem).wait()
        pltpu.make_async_copy(cc_hbm.at[sl], ccv, sem).wait()

    def compact(ci, bufs, g):
        ctraw, utraw, ccv = bufs
        ctc, utc, starts = g[0], g[1], g[2]

        def row_compact(b, off):
            plsc.store_scatter(starts, [_splat(b)], off, mask=lane0)
            nusplat = plsc.load_gather(nuv, [_splat(ci * CB + b)])
            for gi in range(13):
                l0 = gi * 16
                base = b * L + l0
                lvec = lane + l0
                cc16 = ccv[pl.ds(base, LANES)]
                valid = (lvec < nusplat) & (cc16 == 0)
                if gi == 12:
                    valid = valid & (lane < 8)
                cs = plsc.cumsum(valid.astype(jnp.int32))
                dst = off + cs - 1
                plsc.store_scatter(ctc, [dst], ctraw[pl.ds(base, LANES)],
                                   mask=valid)
                plsc.store_scatter(utc, [dst], utraw[pl.ds(base, LANES)],
                                   mask=valid)
                off = off + plsc.all_reduce_population_count(valid)
            odd = off & 1
            pad = lane0 & (odd == 1)
            plsc.store_scatter(ctc, [off], _splat(ZERO_IDX), mask=pad)
            plsc.store_scatter(utc, [off], _splat(0), mask=pad)
            return off + odd

        off = lax.fori_loop(0, CB, row_compact,
                            jnp.zeros((LANES,), jnp.int32))
        plsc.store_scatter(starts, [_splat(CB)], off, mask=lane0)

    def counts(g):
        total = _to_scalar(plsc.load_gather(g[2], [_splat(CB)]))
        nfull = total // WIN
        ntail = (total - nfull * WIN + 31) // 32
        return nfull, ntail

    def fire_g(g, semg):
        ctc, utc, ctrows, utrows = g[0], g[1], g[3], g[4]
        nfull, ntail = counts(g)

        @pl.loop(0, nfull)
        def _fire(j):
            sl = pl.ds(j * WIN, WIN)
            pltpu.async_copy(ctw_hbm.at[ctc.at[sl]], ctrows.at[sl], semg)
            pltpu.async_copy(utw_hbm.at[utc.at[sl]], utrows.at[sl], semg)

        @pl.loop(0, ntail)
        def _firet(j):
            sl = pl.ds(nfull * WIN + j * 32, 32)
            pltpu.async_copy(ctw_hbm.at[ctc.at[sl]], ctrows.at[sl], semg)
            pltpu.async_copy(utw_hbm.at[utc.at[sl]], utrows.at[sl], semg)

    def drain_g(g, semg):
        ctc, utc, ctrows, utrows = g[0], g[1], g[3], g[4]
        nfull, ntail = counts(g)

        @pl.loop(0, nfull)
        def _drain(j):
            sl = pl.ds(j * WIN, WIN)
            pltpu.make_async_copy(ctw_hbm.at[ctc.at[sl]],
                                  ctrows.at[sl], semg).wait()
            pltpu.make_async_copy(utw_hbm.at[utc.at[sl]],
                                  utrows.at[sl], semg).wait()

        @pl.loop(0, ntail)
        def _draint(j):
            sl = pl.ds(nfull * WIN + j * 32, 32)
            pltpu.make_async_copy(ctw_hbm.at[ctc.at[sl]],
                                  ctrows.at[sl], semg).wait()
            pltpu.make_async_copy(utw_hbm.at[utc.at[sl]],
                                  utrows.at[sl], semg).wait()

    def accum_out(ci, g, outv, semo):
        starts, ctrows, utrows = g[2], g[3], g[4]

        @pl.when(ci >= 2)
        def _drain_out():
            back = pl.ds(row0 + (ci - 2) * CB, CB)
            pltpu.make_async_copy(outv, out_hbm.at[back], semo).wait()

        @pl.loop(0, CB)
        def _accum(b):
            s = _to_scalar(plsc.load_gather(starts, [_splat(b)]))
            e = _to_scalar(plsc.load_gather(starts, [_splat(b + 1)]))

            def lbody(i, accs):
                a0, a1 = accs
                la = 2 * i
                ca = ctrows[la, pl.ds(0, 2 * LANES)]
                ua = utrows[la, pl.ds(0, 2 * LANES)]
                cb = ctrows[la + 1, pl.ds(0, 2 * LANES)]
                ub = utrows[la + 1, pl.ds(0, 2 * LANES)]
                pa, pb = plsc.unpack(ca * ua,
                                     format=plsc.PackFormat.INTERLEAVED)
                qa, qb = plsc.unpack(cb * ub,
                                     format=plsc.PackFormat.INTERLEAVED)
                return (a0 + pa + qa, a1 + pb + qb)

            z = jnp.zeros((LANES,), jnp.float32)
            a0, a1 = lax.fori_loop(s // 2, e // 2, lbody, (z, z))
            outv[b, pl.ds(0, LANES)] = a0
            outv[b, pl.ds(LANES, LANES)] = a1

        pltpu.async_copy(outv, out_hbm.at[pl.ds(row0 + ci * CB, CB)], semo)

    i0 = (ctraw0, utraw0, ccv0)
    i1 = (ctraw1, utraw1, ccv1)
    g0 = (ctc0, utc0, starts0, ctrows0, utrows0)
    g1 = (ctc1, utc1, starts1, ctrows1, utrows1)

    fire_idx(0, i0, semi0)
    fire_idx(1, i1, semi1)
    wait_idx(0, i0, semi0)
    compact(0, i0, g0)
    fire_g(g0, semg0)

    @pl.loop(0, NCHUNK, step=2)
    def _chunks(ci):
        wait_idx(ci + 1, i1, semi1)
        compact(ci + 1, i1, g1)
        fire_g(g1, semg1)
        drain_g(g0, semg0)

        @pl.when(ci + 2 < NCHUNK)
        def _pf0():
            fire_idx(ci + 2, i0, semi0)

        accum_out(ci, g0, outv0, semo0)

        @pl.when(ci + 2 < NCHUNK)
        def _next0():
            wait_idx(ci + 2, i0, semi0)
            compact(ci + 2, i0, g0)
            fire_g(g0, semg0)

        drain_g(g1, semg1)

        @pl.when(ci + 3 < NCHUNK)
        def _pf1():
            fire_idx(ci + 3, i1, semi1)

        accum_out(ci + 1, g1, outv1, semo1)

    lastrow0 = pl.ds(row0 + (NCHUNK - 2) * CB, CB)
    lastrow1 = pl.ds(row0 + (NCHUNK - 1) * CB, CB)
    pltpu.make_async_copy(outv0, out_hbm.at[lastrow0], semo0).wait()
    pltpu.make_async_copy(outv1, out_hbm.at[lastrow1], semo1).wait()


def kernel(ctype, ccont, utype, num_unit, ctype_w, utype_w, ccont_w):
    del ccont_w
    ct1 = ctype.reshape(BL)
    ut1 = utype.reshape(BL)
    cc1 = ccont.reshape(BL)
    perm = jnp.arange(D).reshape(2, D // 2).T.reshape(D)
    ctw = jnp.concatenate(
        [ctype_w[:, perm], jnp.zeros((8, D), ctype_w.dtype)]
    ).astype(jnp.bfloat16)
    utw = utype_w[:, perm].astype(jnp.bfloat16)

    mesh = plsc.VectorSubcoreMesh(
        core_axis_name="c", subcore_axis_name="s",
        num_cores=NCORES, num_subcores=NSUB)
    cp = pltpu.CompilerParams(
        needs_layout_passes=False, use_tc_tiling_on_sc=False)
    run = pl.kernel(
        _sc_body,
        out_type=jax.ShapeDtypeStruct((B, D), jnp.float32),
        mesh=mesh,
        compiler_params=cp,
        scratch_types=[
            pltpu.VMEM((IDX_N,), jnp.int32),
            pltpu.VMEM((IDX_N,), jnp.int32),
            pltpu.VMEM((IDX_N,), jnp.int32),
            pltpu.VMEM((IDX_N,), jnp.int32),
            pltpu.VMEM((IDX_N,), jnp.int32),
            pltpu.VMEM((IDX_N,), jnp.int32),
            pltpu.VMEM((RPW,), jnp.int32),
            pltpu.VMEM((CAP,), jnp.int32),
            pltpu.VMEM((CAP,), jnp.int32),
            pltpu.VMEM((16,), jnp.int32),
            pltpu.VMEM((CAP, D), jnp.bfloat16),
            pltpu.VMEM((CAP, D), jnp.bfloat16),
            pltpu.VMEM((CAP,), jnp.int32),
            pltpu.VMEM((CAP,), jnp.int32),
            pltpu.VMEM((16,), jnp.int32),
            pltpu.VMEM((CAP, D), jnp.bfloat16),
            pltpu.VMEM((CAP, D), jnp.bfloat16),
            pltpu.VMEM((CB, D), jnp.float32),
            pltpu.VMEM((CB, D), jnp.float32),
            pltpu.SemaphoreType.DMA,
            pltpu.SemaphoreType.DMA,
            pltpu.SemaphoreType.DMA,
            pltpu.SemaphoreType.DMA,
            pltpu.SemaphoreType.DMA,
            pltpu.SemaphoreType.DMA,
        ],
    )
    return run(ct1, ut1, cc1, num_unit, ctw, utw)
